# baseline (device time: 39320 ns/iter reference)
import jax
import jax.numpy as jnp
from jax import lax
from jax.experimental import pallas as pl
from jax.experimental.pallas import tpu as pltpu

N_DEV = 4
SCALE = 0.08838834764831843
BLK = 64
H_PER_STEP = 2


def _kv_copy(hbm_ref, buf_ref, sems, head, slot):
    return pltpu.make_async_copy(
        hbm_ref.at[0, :, head, :], buf_ref.at[slot], sems.at[slot]
    )


def _ctx(xs, wq_h, k, v, bias):
    q = jnp.dot(
        xs, wq_h, preferred_element_type=jnp.float32
    ).astype(jnp.bfloat16)
    scores = lax.dot_general(
        q, k.astype(jnp.bfloat16), (((1,), (1,)), ((), ())),
        preferred_element_type=jnp.float32,
    ).astype(jnp.bfloat16) + bias
    e = jnp.exp(scores)
    denom = jnp.sum(e, axis=1, keepdims=True, dtype=jnp.float32)
    ctx = lax.dot_general(
        e, v.astype(jnp.bfloat16),
        (((1,), (0,)), ((), ())), preferred_element_type=jnp.float32,
    )
    return (ctx / denom).astype(jnp.bfloat16)


def _body(x_ref, wq_ref, k_hbm, v_hbm, wo_ref, out_ref,
          acc_ref, bias_ref, xs_ref, ka_buf, kb_buf, va_buf, vb_buf,
          ka_sems, kb_sems, va_sems, vb_sems,
          comm_ref, sb_ref, send_sems, recv_sems):
    j = pl.program_id(0)
    nj = pl.num_programs(0)
    slot = j % 2
    nslot = (j + 1) % 2
    ha = j * H_PER_STEP
    hb = ha + 1

    sq = x_ref.shape[1]
    skv = k_hbm.shape[1]
    dh = k_hbm.shape[3]
    dm = x_ref.shape[2]
    half = dm // 2

    my = lax.axis_index("i")
    p1 = my ^ 1
    p2 = 3 - my

    def _xchg(stage_slot, partner):
        return pltpu.make_async_remote_copy(
            src_ref=sb_ref.at[stage_slot],
            dst_ref=comm_ref.at[stage_slot],
            send_sem=send_sems.at[stage_slot],
            recv_sem=recv_sems.at[stage_slot],
            device_id=(partner,),
            device_id_type=pl.DeviceIdType.MESH,
        )

    @pl.when(j == 0)
    def _():
        _kv_copy(k_hbm, ka_buf, ka_sems, ha, 0).start()
        _kv_copy(k_hbm, kb_buf, kb_sems, hb, 0).start()
        _kv_copy(v_hbm, va_buf, va_sems, ha, 0).start()
        _kv_copy(v_hbm, vb_buf, vb_sems, hb, 0).start()
        barrier_sem = pltpu.get_barrier_semaphore()
        for nbr in (p1, p2):
            pl.semaphore_signal(
                barrier_sem, inc=1,
                device_id=(nbr,), device_id_type=pl.DeviceIdType.MESH,
            )
        pl.semaphore_wait(barrier_sem, 2)
        qb = lax.broadcasted_iota(jnp.int32, (sq, skv), 0) // BLK
        kb = lax.broadcasted_iota(jnp.int32, (sq, skv), 1) // BLK
        mask = (qb == kb) | (kb == 0) | ((qb + kb) % 3 == 0)
        bias_ref[...] = jnp.where(mask, 0.0, -1e9).astype(jnp.bfloat16)
        xs_ref[...] = (x_ref[0] * SCALE).astype(jnp.bfloat16)

    _kv_copy(k_hbm, ka_buf, ka_sems, ha, slot).wait()
    _kv_copy(k_hbm, kb_buf, kb_sems, hb, slot).wait()
    _kv_copy(v_hbm, va_buf, va_sems, ha, slot).wait()
    _kv_copy(v_hbm, vb_buf, vb_sems, hb, slot).wait()

    @pl.when(j + 1 < nj)
    def _():
        _kv_copy(k_hbm, ka_buf, ka_sems, ha + H_PER_STEP, nslot).start()
        _kv_copy(k_hbm, kb_buf, kb_sems, hb + H_PER_STEP, nslot).start()
        _kv_copy(v_hbm, va_buf, va_sems, ha + H_PER_STEP, nslot).start()
        _kv_copy(v_hbm, vb_buf, vb_sems, hb + H_PER_STEP, nslot).start()

    xs = xs_ref[...]
    wq = wq_ref[...].astype(jnp.bfloat16)
    wo = wo_ref[...].astype(jnp.bfloat16)
    bias = bias_ref[...]

    ctx_a = _ctx(xs, wq[:, :dh], ka_buf[slot], va_buf[slot], bias)
    ctx_b = _ctx(xs, wq[:, dh:], kb_buf[slot], vb_buf[slot], bias)

    @pl.when(j == 0)
    def _():
        acc_ref[...] = (
            jnp.dot(ctx_a, wo[:dh, :], preferred_element_type=jnp.float32)
            + jnp.dot(ctx_b, wo[dh:, :], preferred_element_type=jnp.float32)
        )

    @pl.when((j != 0) & (j != nj - 1))
    def _():
        acc_ref[...] += (
            jnp.dot(ctx_a, wo[:dh, :], preferred_element_type=jnp.float32)
            + jnp.dot(ctx_b, wo[dh:, :], preferred_element_type=jnp.float32)
        )

    @pl.when(j == nj - 1)
    def _():
        acc_l = acc_ref[:, :half] + (
            jnp.dot(ctx_a, wo[:dh, :half], preferred_element_type=jnp.float32)
            + jnp.dot(ctx_b, wo[dh:, :half], preferred_element_type=jnp.float32)
        )
        sb_ref[0] = acc_l.astype(jnp.bfloat16)
        r1l = _xchg(0, p1)
        r1l.start()

        acc_r = acc_ref[:, half:] + (
            jnp.dot(ctx_a, wo[:dh, half:], preferred_element_type=jnp.float32)
            + jnp.dot(ctx_b, wo[dh:, half:], preferred_element_type=jnp.float32)
        )
        sb_ref[1] = acc_r.astype(jnp.bfloat16)
        r1r = _xchg(1, p1)
        r1r.start()

        r1l.wait()
        acc1_l = acc_l + comm_ref[0].astype(jnp.float32)
        sb_ref[2] = acc1_l.astype(jnp.bfloat16)
        r2l = _xchg(2, p2)
        r2l.start()

        r1r.wait()
        acc1_r = acc_r + comm_ref[1].astype(jnp.float32)
        sb_ref[3] = acc1_r.astype(jnp.bfloat16)
        r2r = _xchg(3, p2)
        r2r.start()

        r2l.wait()
        out_ref[0, :, :half] = (
            acc1_l + comm_ref[2].astype(jnp.float32)
        ).astype(jnp.bfloat16)
        r2r.wait()
        out_ref[0, :, half:] = (
            acc1_r + comm_ref[3].astype(jnp.float32)
        ).astype(jnp.bfloat16)


def kernel(x, Wq, K_ext, V_ext, Wo):
    b, sq, dm = x.shape
    _, skv, h_local, dh = K_ext.shape
    n_steps = h_local // H_PER_STEP

    def _pair(j):
        return lax.axis_index("i") * n_steps + j

    return pl.pallas_call(
        _body,
        grid=(n_steps,),
        out_shape=jax.ShapeDtypeStruct((b, sq, dm), jnp.bfloat16),
        in_specs=[
            pl.BlockSpec((b, sq, dm), lambda j: (0, 0, 0)),
            pl.BlockSpec((dm, H_PER_STEP * dh), lambda j: (0, _pair(j))),
            pl.BlockSpec(memory_space=pl.ANY),
            pl.BlockSpec(memory_space=pl.ANY),
            pl.BlockSpec((H_PER_STEP * dh, dm), lambda j: (_pair(j), 0)),
        ],
        out_specs=pl.BlockSpec((b, sq, dm), lambda j: (0, 0, 0)),
        scratch_shapes=[
            pltpu.VMEM((sq, dm), jnp.float32),
            pltpu.VMEM((sq, skv), jnp.bfloat16),
            pltpu.VMEM((sq, dm), jnp.bfloat16),
            pltpu.VMEM((2, skv, dh), jnp.float32),
            pltpu.VMEM((2, skv, dh), jnp.float32),
            pltpu.VMEM((2, skv, dh), jnp.float32),
            pltpu.VMEM((2, skv, dh), jnp.float32),
            pltpu.SemaphoreType.DMA((2,)),
            pltpu.SemaphoreType.DMA((2,)),
            pltpu.SemaphoreType.DMA((2,)),
            pltpu.SemaphoreType.DMA((2,)),
            pltpu.VMEM((4, sq, dm // 2), jnp.bfloat16),
            pltpu.VMEM((4, sq, dm // 2), jnp.bfloat16),
            pltpu.SemaphoreType.DMA((4,)),
            pltpu.SemaphoreType.DMA((4,)),
        ],
        compiler_params=pltpu.CompilerParams(
            collective_id=0, vmem_limit_bytes=56 * 1024 * 1024
        ),
    )(x, Wq, K_ext, V_ext, Wo)


# device time: 36169 ns/iter; 1.0871x vs baseline; 1.0871x over previous
import jax
import jax.numpy as jnp
from jax import lax
from jax.experimental import pallas as pl
from jax.experimental.pallas import tpu as pltpu

N_DEV = 4
SCALE = 0.08838834764831843
BLK = 64
H_PER_STEP = 2


def _kv_copy(hbm_ref, buf_ref, sems, head, slot):
    return pltpu.make_async_copy(
        hbm_ref.at[0, :, head, :], buf_ref.at[slot], sems.at[slot]
    )


def _ctx(qs, k, v, bias):
    scores = lax.dot_general(
        qs, k.astype(jnp.bfloat16), (((1,), (1,)), ((), ())),
        preferred_element_type=jnp.float32,
    ) + bias
    e = jnp.exp(scores)
    denom = jnp.sum(e, axis=1, keepdims=True)
    ctx = lax.dot_general(
        e.astype(jnp.bfloat16), v.astype(jnp.bfloat16),
        (((1,), (0,)), ((), ())), preferred_element_type=jnp.float32,
    )
    return (ctx / denom).astype(jnp.bfloat16)


def _body(x_ref, wq_ref, k_hbm, v_hbm, wo_ref, out_ref,
          acc_ref, bias_ref, ka_buf, kb_buf, va_buf, vb_buf,
          ka_sems, kb_sems, va_sems, vb_sems,
          comm_ref, sb_ref, send_sems, recv_sems):
    j = pl.program_id(0)
    nj = pl.num_programs(0)
    slot = j % 2
    nslot = (j + 1) % 2
    ha = j * H_PER_STEP
    hb = ha + 1

    sq = x_ref.shape[1]
    skv = k_hbm.shape[1]
    dh = k_hbm.shape[3]
    dm = x_ref.shape[2]
    half = dm // 2

    my = lax.axis_index("i")
    p1 = my ^ 1
    p2 = 3 - my

    def _xchg(stage_slot, partner):
        return pltpu.make_async_remote_copy(
            src_ref=sb_ref.at[stage_slot],
            dst_ref=comm_ref.at[stage_slot],
            send_sem=send_sems.at[stage_slot],
            recv_sem=recv_sems.at[stage_slot],
            device_id=(partner,),
            device_id_type=pl.DeviceIdType.MESH,
        )

    @pl.when(j == 0)
    def _():
        _kv_copy(k_hbm, ka_buf, ka_sems, ha, 0).start()
        _kv_copy(k_hbm, kb_buf, kb_sems, hb, 0).start()
        _kv_copy(v_hbm, va_buf, va_sems, ha, 0).start()
        _kv_copy(v_hbm, vb_buf, vb_sems, hb, 0).start()
        barrier_sem = pltpu.get_barrier_semaphore()
        for nbr in (p1, p2):
            pl.semaphore_signal(
                barrier_sem, inc=1,
                device_id=(nbr,), device_id_type=pl.DeviceIdType.MESH,
            )
        pl.semaphore_wait(barrier_sem, 2)
        qb = lax.broadcasted_iota(jnp.int32, (sq, skv), 0) // BLK
        kb = lax.broadcasted_iota(jnp.int32, (sq, skv), 1) // BLK
        mask = (qb == kb) | (kb == 0) | ((qb + kb) % 3 == 0)
        bias_ref[...] = jnp.where(mask, 0.0, -1e9).astype(jnp.float32)

    _kv_copy(k_hbm, ka_buf, ka_sems, ha, slot).wait()
    _kv_copy(k_hbm, kb_buf, kb_sems, hb, slot).wait()
    _kv_copy(v_hbm, va_buf, va_sems, ha, slot).wait()
    _kv_copy(v_hbm, vb_buf, vb_sems, hb, slot).wait()

    @pl.when(j + 1 < nj)
    def _():
        _kv_copy(k_hbm, ka_buf, ka_sems, ha + H_PER_STEP, nslot).start()
        _kv_copy(k_hbm, kb_buf, kb_sems, hb + H_PER_STEP, nslot).start()
        _kv_copy(v_hbm, va_buf, va_sems, ha + H_PER_STEP, nslot).start()
        _kv_copy(v_hbm, vb_buf, vb_sems, hb + H_PER_STEP, nslot).start()

    xm = x_ref[0].astype(jnp.bfloat16)
    wq = wq_ref[...].astype(jnp.bfloat16)
    wo = wo_ref[...].astype(jnp.bfloat16)
    bias = bias_ref[...]

    q2 = jnp.dot(xm, wq, preferred_element_type=jnp.float32)
    qs2 = (q2 * SCALE).astype(jnp.bfloat16)

    ctx_a = _ctx(qs2[:, :dh], ka_buf[slot], va_buf[slot], bias)
    ctx_b = _ctx(qs2[:, dh:], kb_buf[slot], vb_buf[slot], bias)

    @pl.when(j == 0)
    def _():
        acc_ref[...] = (
            jnp.dot(ctx_a, wo[:dh, :], preferred_element_type=jnp.float32)
            + jnp.dot(ctx_b, wo[dh:, :], preferred_element_type=jnp.float32)
        )

    @pl.when((j != 0) & (j != nj - 1))
    def _():
        acc_ref[...] += (
            jnp.dot(ctx_a, wo[:dh, :], preferred_element_type=jnp.float32)
            + jnp.dot(ctx_b, wo[dh:, :], preferred_element_type=jnp.float32)
        )

    @pl.when(j == nj - 1)
    def _():
        acc_l = acc_ref[:, :half] + (
            jnp.dot(ctx_a, wo[:dh, :half], preferred_element_type=jnp.float32)
            + jnp.dot(ctx_b, wo[dh:, :half], preferred_element_type=jnp.float32)
        )
        sb_ref[0] = acc_l.astype(jnp.bfloat16)
        r1l = _xchg(0, p1)
        r1l.start()

        acc_r = acc_ref[:, half:] + (
            jnp.dot(ctx_a, wo[:dh, half:], preferred_element_type=jnp.float32)
            + jnp.dot(ctx_b, wo[dh:, half:], preferred_element_type=jnp.float32)
        )
        sb_ref[1] = acc_r.astype(jnp.bfloat16)
        r1r = _xchg(1, p2)
        r1r.start()

        r1l.wait()
        acc1_l = acc_l + comm_ref[0].astype(jnp.float32)
        sb_ref[2] = acc1_l.astype(jnp.bfloat16)
        r2l = _xchg(2, p2)
        r2l.start()

        r1r.wait()
        acc1_r = acc_r + comm_ref[1].astype(jnp.float32)
        sb_ref[3] = acc1_r.astype(jnp.bfloat16)
        r2r = _xchg(3, p1)
        r2r.start()

        r2l.wait()
        out_ref[0, :, :half] = (
            acc1_l + comm_ref[2].astype(jnp.float32)
        ).astype(jnp.bfloat16)
        r2r.wait()
        out_ref[0, :, half:] = (
            acc1_r + comm_ref[3].astype(jnp.float32)
        ).astype(jnp.bfloat16)


def kernel(x, Wq, K_ext, V_ext, Wo):
    b, sq, dm = x.shape
    _, skv, h_local, dh = K_ext.shape
    n_steps = h_local // H_PER_STEP

    def _pair(j):
        return lax.axis_index("i") * n_steps + j

    return pl.pallas_call(
        _body,
        grid=(n_steps,),
        out_shape=jax.ShapeDtypeStruct((b, sq, dm), jnp.bfloat16),
        in_specs=[
            pl.BlockSpec((b, sq, dm), lambda j: (0, 0, 0)),
            pl.BlockSpec((dm, H_PER_STEP * dh), lambda j: (0, _pair(j))),
            pl.BlockSpec(memory_space=pl.ANY),
            pl.BlockSpec(memory_space=pl.ANY),
            pl.BlockSpec((H_PER_STEP * dh, dm), lambda j: (_pair(j), 0)),
        ],
        out_specs=pl.BlockSpec((b, sq, dm), lambda j: (0, 0, 0)),
        scratch_shapes=[
            pltpu.VMEM((sq, dm), jnp.float32),
            pltpu.VMEM((sq, skv), jnp.float32),
            pltpu.VMEM((2, skv, dh), jnp.float32),
            pltpu.VMEM((2, skv, dh), jnp.float32),
            pltpu.VMEM((2, skv, dh), jnp.float32),
            pltpu.VMEM((2, skv, dh), jnp.float32),
            pltpu.SemaphoreType.DMA((2,)),
            pltpu.SemaphoreType.DMA((2,)),
            pltpu.SemaphoreType.DMA((2,)),
            pltpu.SemaphoreType.DMA((2,)),
            pltpu.VMEM((4, sq, dm // 2), jnp.bfloat16),
            pltpu.VMEM((4, sq, dm // 2), jnp.bfloat16),
            pltpu.SemaphoreType.DMA((4,)),
            pltpu.SemaphoreType.DMA((4,)),
        ],
        compiler_params=pltpu.CompilerParams(
            collective_id=0, vmem_limit_bytes=56 * 1024 * 1024
        ),
    )(x, Wq, K_ext, V_ext, Wo)
